# baseline (speedup 1.0000x reference)
import functools

import jax
import jax.numpy as jnp
from jax import lax
from jax.experimental import pallas as pl
from jax.experimental.pallas import tpu as pltpu
from jax.experimental.pallas import tpu_sc as plsc

_H = 768
_F = 1536
_E = 8
_T = 2048
_P = 2 * _T
_BM = 256
_PPAD = _P + _E * _BM
_NB = _PPAD // _BM

_NC = 2
_NS = 16
_NW = _NC * _NS
_L = 16

_HW = _H // 2
_RPW = _PPAD // _NW
_GCH = 64
_TPW = _T // _NW
_TCH = 16


def _router_body(x_ref, gwt_ref, topi_ref, topw_ref):
    x = x_ref[...]
    gwt = gwt_ref[...]
    logits = jnp.dot(x, gwt, preferred_element_type=jnp.float32)
    iota = lax.broadcasted_iota(jnp.int32, logits.shape, 1)
    m1 = jnp.max(logits, axis=1, keepdims=True)
    i1 = jnp.min(jnp.where(logits == m1, iota, _E), axis=1, keepdims=True)
    l2 = jnp.where(iota == i1, -jnp.inf, logits)
    m2 = jnp.max(l2, axis=1, keepdims=True)
    i2 = jnp.min(jnp.where(l2 == m2, iota, _E), axis=1, keepdims=True)
    t = jnp.exp(m2 - m1)
    wa = 1.0 / (1.0 + t)
    wb = t / (1.0 + t)
    topi_ref[...] = jnp.concatenate([i1, i2], axis=1)
    topw_ref[...] = jnp.concatenate([wa, wb], axis=1)


def _gemm_body(be_ref, xs_ref, w1_ref, w2_ref, ys_ref, w1c, w2c):
    b = pl.program_id(0)
    fresh = (b == 0) | (be_ref[b] != be_ref[jnp.maximum(b - 1, 0)])

    @pl.when(fresh)
    def _():
        w1c[...] = w1_ref[0].astype(jnp.bfloat16)
        w2c[...] = w2_ref[0].astype(jnp.bfloat16)

    dn = (((1,), (1,)), ((), ()))
    h = lax.dot_general(xs_ref[...].astype(jnp.bfloat16), w1c[...], dn,
                        preferred_element_type=jnp.float32)
    h = (h * jax.nn.sigmoid(h)).astype(jnp.bfloat16)
    o = lax.dot_general(h, w2c[...], dn, preferred_element_type=jnp.float32)
    ys_ref[...] = o


_SC_MESH = plsc.VectorSubcoreMesh(core_axis_name="c", subcore_axis_name="s")


def _dispatch_body(x_hbm, pe_hbm, po_hbm, xs_hbm, idx_e, idx_o, rows_v,
                   gs, se0, se1):
    wid = lax.axis_index("s") * _NC + lax.axis_index("c")
    tb = wid * _TPW
    pltpu.sync_copy(pe_hbm.at[pl.ds(tb, _TPW)], idx_e)
    pltpu.sync_copy(po_hbm.at[pl.ds(tb, _TPW)], idx_o)
    g = pltpu.async_copy(x_hbm.at[pl.ds(tb, _TPW)], rows_v, gs)
    g.wait()
    s0 = pltpu.async_copy(rows_v, xs_hbm.at[idx_e], se0)
    s1 = pltpu.async_copy(rows_v, xs_hbm.at[idx_o], se1)
    s0.wait()
    s1.wait()


def _weighted_pair_add(rows, out, wv, poff, n_tok):
    def tok(t, carry):
        wa = wv[poff + 2 * t, :]
        wb = wv[poff + 2 * t + 1, :]
        for j in range(_H // _L):
            sl = pl.ds(j * _L, _L)
            out[t, sl] = wa * rows[2 * t, sl] + wb * rows[2 * t + 1, sl]
        return carry

    lax.fori_loop(0, n_tok, tok, 0)


def _combine_body(ys_hbm, pos_hbm, wsp_hbm, out_hbm, idx_v, wv_v,
                  rows0, rows1, out0, out1, gs0, gs1, ss0, ss1):
    wid = lax.axis_index("s") * _NC + lax.axis_index("c")
    tbase = wid * _TPW
    pltpu.sync_copy(pos_hbm.at[pl.ds(2 * tbase, 2 * _TPW)], idx_v)
    pltpu.sync_copy(wsp_hbm.at[pl.ds(2 * tbase, 2 * _TPW)], wv_v)
    rows = [rows0, rows1]
    outs = [out0, out1]
    gsems = [gs0, gs1]
    ssems = [ss0, ss1]
    nch = _TPW // _TCH
    gathers = {}
    stores = {}
    for c in range(2):
        gathers[c] = pltpu.async_copy(
            ys_hbm.at[idx_v.at[pl.ds(c * 2 * _TCH, 2 * _TCH)]],
            rows[c], gsems[c])
    for c in range(nch):
        b = c % 2
        gathers[c].wait()
        if c >= 2:
            stores[c - 2].wait()
        _weighted_pair_add(rows[b], outs[b], wv_v, c * 2 * _TCH, _TCH)
        stores[c] = pltpu.async_copy(
            outs[b], out_hbm.at[pl.ds(tbase + c * _TCH, _TCH)], ssems[b])
        if c + 2 < nch:
            gathers[c + 2] = pltpu.async_copy(
                ys_hbm.at[idx_v.at[pl.ds((c + 2) * 2 * _TCH, 2 * _TCH)]],
                rows[b], gsems[b])
    stores[nch - 2].wait()
    stores[nch - 1].wait()


_sc_dispatch = pl.kernel(
    _dispatch_body,
    out_type=jax.ShapeDtypeStruct((_PPAD, _H), jnp.float32),
    mesh=_SC_MESH,
    scratch_types=[
        pltpu.VMEM((_TPW,), jnp.int32),
        pltpu.VMEM((_TPW,), jnp.int32),
        pltpu.VMEM((_TPW, _H), jnp.float32),
        pltpu.SemaphoreType.DMA,
        pltpu.SemaphoreType.DMA,
        pltpu.SemaphoreType.DMA,
    ],
)


_sc_combine = pl.kernel(
    _combine_body,
    out_type=jax.ShapeDtypeStruct((_T, _H), jnp.float32),
    mesh=_SC_MESH,
    scratch_types=[
        pltpu.VMEM((2 * _TPW,), jnp.int32),
        pltpu.VMEM((2 * _TPW, _L), jnp.float32),
        pltpu.VMEM((2 * _TCH, _H), jnp.float32),
        pltpu.VMEM((2 * _TCH, _H), jnp.float32),
        pltpu.VMEM((_TCH, _H), jnp.float32),
        pltpu.VMEM((_TCH, _H), jnp.float32),
        pltpu.SemaphoreType.DMA,
        pltpu.SemaphoreType.DMA,
        pltpu.SemaphoreType.DMA,
        pltpu.SemaphoreType.DMA,
    ],
)


def kernel(hidden_states, gate_w, w1, w2):
    bsz, seqlen, hd = hidden_states.shape
    x2 = hidden_states.reshape(-1, hd)
    gwt = gate_w.T

    topi, topw = pl.pallas_call(
        _router_body,
        out_shape=[
            jax.ShapeDtypeStruct((_T, 2), jnp.int32),
            jax.ShapeDtypeStruct((_T, 2), jnp.float32),
        ],
    )(x2, gwt)

    e = topi.reshape(_P)
    wflat = topw.reshape(_P)
    onehot = (e[:, None] == jnp.arange(_E)[None, :]).astype(jnp.int32)
    cum = jnp.cumsum(onehot, axis=0)
    counts = cum[-1]
    rank = jnp.sum(onehot * cum, axis=1) - 1
    padded_counts = ((counts + _BM - 1) // _BM) * _BM
    padded_off = jnp.concatenate(
        [jnp.zeros((1,), jnp.int32), jnp.cumsum(padded_counts)[:-1]])
    pos = jnp.sum(onehot * padded_off[None, :], axis=1) + rank
    pos2 = pos.reshape(_T, 2)
    wsplat = jnp.broadcast_to(wflat[:, None], (_P, _L))
    block_expert = jnp.minimum(
        jnp.searchsorted(jnp.cumsum(padded_counts),
                         jnp.arange(_NB) * _BM, side='right'),
        _E - 1).astype(jnp.int32)

    xs = _sc_dispatch(x2, pos2[:, 0], pos2[:, 1])

    ys = pl.pallas_call(
        _gemm_body,
        grid_spec=pltpu.PrefetchScalarGridSpec(
            num_scalar_prefetch=1,
            grid=(_NB,),
            in_specs=[
                pl.BlockSpec((_BM, _H), lambda b, be: (b, 0)),
                pl.BlockSpec((1, _F, _H), lambda b, be: (be[b], 0, 0)),
                pl.BlockSpec((1, _H, _F), lambda b, be: (be[b], 0, 0)),
            ],
            out_specs=pl.BlockSpec((_BM, _H), lambda b, be: (b, 0)),
            scratch_shapes=[
                pltpu.VMEM((_F, _H), jnp.bfloat16),
                pltpu.VMEM((_H, _F), jnp.bfloat16),
            ],
        ),
        out_shape=jax.ShapeDtypeStruct((_PPAD, _H), jnp.float32),
    )(block_expert, xs, w1, w2)

    out = _sc_combine(ys, pos, wsplat)
    return out.reshape(bsz, seqlen, hd)


# speedup vs baseline: 1.0426x; 1.0426x over previous
import functools

import jax
import jax.numpy as jnp
from jax import lax
from jax.experimental import pallas as pl
from jax.experimental.pallas import tpu as pltpu
from jax.experimental.pallas import tpu_sc as plsc

_H = 768
_F = 1536
_E = 8
_T = 2048
_P = 2 * _T
_BM = 256
_PPAD = _P + _E * _BM
_NB = _PPAD // _BM

_NC = 2
_NS = 16
_NW = _NC * _NS
_L = 16

_HW = _H // 2
_RPW = _PPAD // _NW
_GCH = 64
_TPW = _T // _NW
_TCH = 16


def _router_body(x_ref, gwt_ref, topi_ref, topw_ref, xp_ref):
    x = x_ref[...]
    gwt = gwt_ref[...]
    logits = jnp.dot(x, gwt, preferred_element_type=jnp.float32)
    iota = lax.broadcasted_iota(jnp.int32, logits.shape, 1)
    m1 = jnp.max(logits, axis=1, keepdims=True)
    i1 = jnp.min(jnp.where(logits == m1, iota, _E), axis=1, keepdims=True)
    l2 = jnp.where(iota == i1, -jnp.inf, logits)
    m2 = jnp.max(l2, axis=1, keepdims=True)
    i2 = jnp.min(jnp.where(l2 == m2, iota, _E), axis=1, keepdims=True)
    t = jnp.exp(m2 - m1)
    wa = 1.0 / (1.0 + t)
    wb = t / (1.0 + t)
    topi_ref[...] = jnp.concatenate([i1, i2], axis=1)
    topw_ref[...] = jnp.concatenate([wa, wb], axis=1)
    xb = x.astype(jnp.bfloat16)
    plo = lax.bitcast_convert_type(xb[:, :_HW], jnp.uint16).astype(jnp.uint32)
    phi = lax.bitcast_convert_type(xb[:, _HW:], jnp.uint16).astype(jnp.uint32)
    xp_ref[...] = pltpu.bitcast((phi << 16) | plo, jnp.int32)


def _gemm_body(be_ref, xs_ref, w1_ref, w2_ref, ys_ref):
    xi = pltpu.bitcast(xs_ref[...], jnp.uint32)
    lo = lax.bitcast_convert_type((xi & 0xFFFF).astype(jnp.uint16),
                                  jnp.bfloat16)
    hi = lax.bitcast_convert_type((xi >> 16).astype(jnp.uint16),
                                  jnp.bfloat16)
    xb = jnp.concatenate([lo, hi], axis=1)
    dn = (((1,), (1,)), ((), ()))
    h = lax.dot_general(xb, w1_ref[0].astype(jnp.bfloat16), dn,
                        preferred_element_type=jnp.float32)
    h = (h * jax.nn.sigmoid(h)).astype(jnp.bfloat16)
    o = lax.dot_general(h, w2_ref[0].astype(jnp.bfloat16), dn,
                        preferred_element_type=jnp.float32)
    ys_ref[...] = o


_SC_MESH = plsc.VectorSubcoreMesh(core_axis_name="c", subcore_axis_name="s")


def _dispatch_body(x_hbm, pe_hbm, po_hbm, xs_hbm, idx_e, idx_o, rows_v,
                   gs, se0, se1):
    wid = lax.axis_index("s") * _NC + lax.axis_index("c")
    tb = wid * _TPW
    pltpu.sync_copy(pe_hbm.at[pl.ds(tb, _TPW)], idx_e)
    pltpu.sync_copy(po_hbm.at[pl.ds(tb, _TPW)], idx_o)
    g = pltpu.async_copy(x_hbm.at[pl.ds(tb, _TPW)], rows_v, gs)
    g.wait()
    s0 = pltpu.async_copy(rows_v, xs_hbm.at[idx_e], se0)
    s1 = pltpu.async_copy(rows_v, xs_hbm.at[idx_o], se1)
    s0.wait()
    s1.wait()


def _weighted_pair_add(rows, out, wv, poff, n_tok):
    def tok(t, carry):
        wa = wv[poff + 2 * t, :]
        wb = wv[poff + 2 * t + 1, :]
        for j in range(_H // _L):
            sl = pl.ds(j * _L, _L)
            out[t, sl] = wa * rows[2 * t, sl] + wb * rows[2 * t + 1, sl]
        return carry

    lax.fori_loop(0, n_tok, tok, 0)


def _combine_body(ys_hbm, pos_hbm, wsp_hbm, out_hbm, idx_v, wv_v,
                  rows0, rows1, out0, out1, gs0, gs1, ss0, ss1):
    wid = lax.axis_index("s") * _NC + lax.axis_index("c")
    tbase = wid * _TPW
    pltpu.sync_copy(pos_hbm.at[pl.ds(2 * tbase, 2 * _TPW)], idx_v)
    pltpu.sync_copy(wsp_hbm.at[pl.ds(2 * tbase, 2 * _TPW)], wv_v)
    rows = [rows0, rows1]
    outs = [out0, out1]
    gsems = [gs0, gs1]
    ssems = [ss0, ss1]
    nch = _TPW // _TCH
    gathers = {}
    stores = {}
    for c in range(2):
        gathers[c] = pltpu.async_copy(
            ys_hbm.at[idx_v.at[pl.ds(c * 2 * _TCH, 2 * _TCH)]],
            rows[c], gsems[c])
    for c in range(nch):
        b = c % 2
        gathers[c].wait()
        if c >= 2:
            stores[c - 2].wait()
        _weighted_pair_add(rows[b], outs[b], wv_v, c * 2 * _TCH, _TCH)
        stores[c] = pltpu.async_copy(
            outs[b], out_hbm.at[pl.ds(tbase + c * _TCH, _TCH)], ssems[b])
        if c + 2 < nch:
            gathers[c + 2] = pltpu.async_copy(
                ys_hbm.at[idx_v.at[pl.ds((c + 2) * 2 * _TCH, 2 * _TCH)]],
                rows[b], gsems[b])
    stores[nch - 2].wait()
    stores[nch - 1].wait()


_sc_dispatch = pl.kernel(
    _dispatch_body,
    out_type=jax.ShapeDtypeStruct((_PPAD, _HW), jnp.int32),
    mesh=_SC_MESH,
    scratch_types=[
        pltpu.VMEM((_TPW,), jnp.int32),
        pltpu.VMEM((_TPW,), jnp.int32),
        pltpu.VMEM((_TPW, _HW), jnp.int32),
        pltpu.SemaphoreType.DMA,
        pltpu.SemaphoreType.DMA,
        pltpu.SemaphoreType.DMA,
    ],
)


_sc_combine = pl.kernel(
    _combine_body,
    out_type=jax.ShapeDtypeStruct((_T, _H), jnp.float32),
    mesh=_SC_MESH,
    scratch_types=[
        pltpu.VMEM((2 * _TPW,), jnp.int32),
        pltpu.VMEM((2 * _TPW, _L), jnp.float32),
        pltpu.VMEM((2 * _TCH, _H), jnp.float32),
        pltpu.VMEM((2 * _TCH, _H), jnp.float32),
        pltpu.VMEM((_TCH, _H), jnp.float32),
        pltpu.VMEM((_TCH, _H), jnp.float32),
        pltpu.SemaphoreType.DMA,
        pltpu.SemaphoreType.DMA,
        pltpu.SemaphoreType.DMA,
        pltpu.SemaphoreType.DMA,
    ],
)


def kernel(hidden_states, gate_w, w1, w2):
    bsz, seqlen, hd = hidden_states.shape
    x2 = hidden_states.reshape(-1, hd)
    gwt = gate_w.T

    topi, topw, xp = pl.pallas_call(
        _router_body,
        out_shape=[
            jax.ShapeDtypeStruct((_T, 2), jnp.int32),
            jax.ShapeDtypeStruct((_T, 2), jnp.float32),
            jax.ShapeDtypeStruct((_T, _HW), jnp.int32),
        ],
    )(x2, gwt)

    e = topi.reshape(_P)
    wflat = topw.reshape(_P)
    onehot = (e[:, None] == jnp.arange(_E)[None, :]).astype(jnp.int32)
    cum = jnp.cumsum(onehot, axis=0)
    counts = cum[-1]
    rank = jnp.sum(onehot * cum, axis=1) - 1
    padded_counts = ((counts + _BM - 1) // _BM) * _BM
    padded_off = jnp.concatenate(
        [jnp.zeros((1,), jnp.int32), jnp.cumsum(padded_counts)[:-1]])
    pos = jnp.sum(onehot * padded_off[None, :], axis=1) + rank
    pos2 = pos.reshape(_T, 2)
    wsplat = jnp.broadcast_to(wflat[:, None], (_P, _L))
    block_expert = jnp.minimum(
        jnp.searchsorted(jnp.cumsum(padded_counts),
                         jnp.arange(_NB) * _BM, side='right'),
        _E - 1).astype(jnp.int32)

    xs = _sc_dispatch(xp, pos2[:, 0], pos2[:, 1])

    ys = pl.pallas_call(
        _gemm_body,
        grid_spec=pltpu.PrefetchScalarGridSpec(
            num_scalar_prefetch=1,
            grid=(_NB,),
            in_specs=[
                pl.BlockSpec((_BM, _HW), lambda b, be: (b, 0)),
                pl.BlockSpec((1, _F, _H), lambda b, be: (be[b], 0, 0)),
                pl.BlockSpec((1, _H, _F), lambda b, be: (be[b], 0, 0)),
            ],
            out_specs=pl.BlockSpec((_BM, _H), lambda b, be: (b, 0)),
        ),
        out_shape=jax.ShapeDtypeStruct((_PPAD, _H), jnp.float32),
    )(block_expert, xs, w1, w2)

    out = _sc_combine(ys, pos, wsplat)
    return out.reshape(bsz, seqlen, hd)


# speedup vs baseline: 1.0439x; 1.0012x over previous
import jax
import jax.numpy as jnp
from jax import lax
from jax.experimental import pallas as pl
from jax.experimental.pallas import tpu as pltpu
from jax.experimental.pallas import tpu_sc as plsc

_H = 768
_F = 1536
_E = 8
_T = 2048
_P = 2 * _T
_BM = 256
_PPAD = _P + _E * _BM
_NB = _PPAD // _BM

_NC = 2
_NS = 16
_NW = _NC * _NS
_L = 16

_HW = _H // 2
_RPW = _PPAD // _NW
_GCH = 64
_TPW = _T // _NW
_TCH = 16


def _router_body(x_ref, gwt_ref, topi_ref, topw_ref, xp_ref):
    x = x_ref[...]
    gwt = gwt_ref[...]
    logits = jnp.dot(x, gwt, preferred_element_type=jnp.float32)
    iota = lax.broadcasted_iota(jnp.int32, logits.shape, 1)
    m1 = jnp.max(logits, axis=1, keepdims=True)
    i1 = jnp.min(jnp.where(logits == m1, iota, _E), axis=1, keepdims=True)
    l2 = jnp.where(iota == i1, -jnp.inf, logits)
    m2 = jnp.max(l2, axis=1, keepdims=True)
    i2 = jnp.min(jnp.where(l2 == m2, iota, _E), axis=1, keepdims=True)
    t = jnp.exp(m2 - m1)
    wa = 1.0 / (1.0 + t)
    wb = t / (1.0 + t)
    topi_ref[...] = jnp.concatenate([i1, i2], axis=1)
    topw_ref[...] = jnp.concatenate([wa, wb], axis=1)
    xb = x.astype(jnp.bfloat16)
    plo = lax.bitcast_convert_type(xb[:, :_HW], jnp.uint16).astype(jnp.uint32)
    phi = lax.bitcast_convert_type(xb[:, _HW:], jnp.uint16).astype(jnp.uint32)
    xp_ref[...] = pltpu.bitcast((phi << 16) | plo, jnp.int32)


def _gemm_body(be_ref, xs_ref, w1_ref, w2_ref, ys_ref):
    xi = pltpu.bitcast(xs_ref[...], jnp.uint32)
    lo = lax.bitcast_convert_type((xi & 0xFFFF).astype(jnp.uint16),
                                  jnp.bfloat16)
    hi = lax.bitcast_convert_type((xi >> 16).astype(jnp.uint16),
                                  jnp.bfloat16)
    xb = jnp.concatenate([lo, hi], axis=1)
    dn = (((1,), (1,)), ((), ()))
    h = lax.dot_general(xb, w1_ref[0].astype(jnp.bfloat16), dn,
                        preferred_element_type=jnp.float32)
    h = (h * jax.nn.sigmoid(h)).astype(jnp.bfloat16)
    o = lax.dot_general(h, w2_ref[0].astype(jnp.bfloat16), dn,
                        preferred_element_type=jnp.float32)
    ys_ref[...] = o


_SC_MESH = plsc.VectorSubcoreMesh(core_axis_name="c", subcore_axis_name="s")


def _dispatch_body(x_hbm, pe_hbm, po_hbm, xs_hbm, idx_e, idx_o, rows_v,
                   gs, se0, se1):
    wid = lax.axis_index("s") * _NC + lax.axis_index("c")
    tb = wid * _TPW
    pltpu.sync_copy(pe_hbm.at[pl.ds(tb, _TPW)], idx_e)
    pltpu.sync_copy(po_hbm.at[pl.ds(tb, _TPW)], idx_o)
    g = pltpu.async_copy(x_hbm.at[pl.ds(tb, _TPW)], rows_v, gs)
    g.wait()
    s0 = pltpu.async_copy(rows_v, xs_hbm.at[idx_e], se0)
    s1 = pltpu.async_copy(rows_v, xs_hbm.at[idx_o], se1)
    s0.wait()
    s1.wait()


def _weighted_pair_add(rows, out, wv, poff, n_tok):
    def tok(t, carry):
        wa = wv[poff + 2 * t, :]
        wb = wv[poff + 2 * t + 1, :]
        for j in range(_H // _L):
            sl = pl.ds(j * _L, _L)
            out[t, sl] = wa * rows[2 * t, sl] + wb * rows[2 * t + 1, sl]
        return carry

    lax.fori_loop(0, n_tok, tok, 0)


def _combine_body(ys_hbm, pos_hbm, wsp_hbm, out_hbm, idx_v, wv_v,
                  rows0, rows1, out0, out1, gs0, gs1, ss0, ss1):
    wid = lax.axis_index("s") * _NC + lax.axis_index("c")
    tbase = wid * _TPW
    pltpu.sync_copy(pos_hbm.at[pl.ds(2 * tbase, 2 * _TPW)], idx_v)
    pltpu.sync_copy(wsp_hbm.at[pl.ds(2 * tbase, 2 * _TPW)], wv_v)
    rows = [rows0, rows1]
    outs = [out0, out1]
    gsems = [gs0, gs1]
    ssems = [ss0, ss1]
    nch = _TPW // _TCH
    gathers = {}
    stores = {}
    for c in range(2):
        gathers[c] = pltpu.async_copy(
            ys_hbm.at[idx_v.at[pl.ds(c * 2 * _TCH, 2 * _TCH)]],
            rows[c], gsems[c])
    for c in range(nch):
        b = c % 2
        gathers[c].wait()
        if c >= 2:
            stores[c - 2].wait()
        _weighted_pair_add(rows[b], outs[b], wv_v, c * 2 * _TCH, _TCH)
        stores[c] = pltpu.async_copy(
            outs[b], out_hbm.at[pl.ds(tbase + c * _TCH, _TCH)], ssems[b])
        if c + 2 < nch:
            gathers[c + 2] = pltpu.async_copy(
                ys_hbm.at[idx_v.at[pl.ds((c + 2) * 2 * _TCH, 2 * _TCH)]],
                rows[b], gsems[b])
    stores[nch - 2].wait()
    stores[nch - 1].wait()


_sc_dispatch = pl.kernel(
    _dispatch_body,
    out_type=jax.ShapeDtypeStruct((_PPAD, _HW), jnp.int32),
    mesh=_SC_MESH,
    scratch_types=[
        pltpu.VMEM((_TPW,), jnp.int32),
        pltpu.VMEM((_TPW,), jnp.int32),
        pltpu.VMEM((_TPW, _HW), jnp.int32),
        pltpu.SemaphoreType.DMA,
        pltpu.SemaphoreType.DMA,
        pltpu.SemaphoreType.DMA,
    ],
)


_sc_combine = pl.kernel(
    _combine_body,
    out_type=jax.ShapeDtypeStruct((_T, _H), jnp.float32),
    mesh=_SC_MESH,
    scratch_types=[
        pltpu.VMEM((2 * _TPW,), jnp.int32),
        pltpu.VMEM((2 * _TPW, _L), jnp.float32),
        pltpu.VMEM((2 * _TCH, _H), jnp.float32),
        pltpu.VMEM((2 * _TCH, _H), jnp.float32),
        pltpu.VMEM((_TCH, _H), jnp.float32),
        pltpu.VMEM((_TCH, _H), jnp.float32),
        pltpu.SemaphoreType.DMA,
        pltpu.SemaphoreType.DMA,
        pltpu.SemaphoreType.DMA,
        pltpu.SemaphoreType.DMA,
    ],
)


def kernel(hidden_states, gate_w, w1, w2):
    bsz, seqlen, hd = hidden_states.shape
    x2 = hidden_states.reshape(-1, hd)
    gwt = gate_w.T

    topi, topw, xp = pl.pallas_call(
        _router_body,
        out_shape=[
            jax.ShapeDtypeStruct((_T, 2), jnp.int32),
            jax.ShapeDtypeStruct((_T, 2), jnp.float32),
            jax.ShapeDtypeStruct((_T, _HW), jnp.int32),
        ],
    )(x2, gwt)

    e = topi.reshape(_P)
    wflat = topw.reshape(_P)
    onehot = (e[:, None] == jnp.arange(_E)[None, :]).astype(jnp.int32)
    cum = jnp.cumsum(onehot, axis=0)
    counts = cum[-1]
    rank = jnp.sum(onehot * cum, axis=1) - 1
    padded_counts = ((counts + _BM - 1) // _BM) * _BM
    padded_off = jnp.concatenate(
        [jnp.zeros((1,), jnp.int32), jnp.cumsum(padded_counts)[:-1]])
    pos = jnp.sum(onehot * padded_off[None, :], axis=1) + rank
    pos2 = pos.reshape(_T, 2)
    wsplat = jnp.broadcast_to(wflat[:, None], (_P, _L))
    block_expert = jnp.minimum(
        jnp.searchsorted(jnp.cumsum(padded_counts),
                         jnp.arange(_NB) * _BM, side='right'),
        _E - 1).astype(jnp.int32)

    xs = _sc_dispatch(xp, pos2[:, 0], pos2[:, 1])

    ys = pl.pallas_call(
        _gemm_body,
        grid_spec=pltpu.PrefetchScalarGridSpec(
            num_scalar_prefetch=1,
            grid=(_NB,),
            in_specs=[
                pl.BlockSpec((_BM, _HW), lambda b, be: (b, 0)),
                pl.BlockSpec((1, _F, _H), lambda b, be: (be[b], 0, 0)),
                pl.BlockSpec((1, _H, _F), lambda b, be: (be[b], 0, 0)),
            ],
            out_specs=pl.BlockSpec((_BM, _H), lambda b, be: (b, 0)),
        ),
        out_shape=jax.ShapeDtypeStruct((_PPAD, _H), jnp.float32),
    )(block_expert, xs, w1, w2)

    out = _sc_combine(ys, pos, wsplat)
    return out.reshape(bsz, seqlen, hd)
